# Initial kernel scaffold; baseline (speedup 1.0000x reference)
import functools

import jax
import jax.numpy as jnp
from jax import lax
from jax.experimental import pallas as pl
from jax.experimental.pallas import tpu as pltpu
from jax.experimental.pallas import tpu_sc as plsc

N = 500000
FEAT = 32
B, R, SR, K = 1, 4096, 24, 8
M = B * R * SR * K
NW = 32
ROWS_PER_W = M // NW
G = 512
NG = ROWS_PER_W // G
CH = 128
NCH = G // CH
L = 16


def _bcast(cam_v, k):
    return plsc.load_gather(cam_v, [jnp.full((L,), k, jnp.int32)])


def _sc_body(emb_hbm, xyz_hbm, pidx_hbm, cam_hbm,
             feats_hbm, pers_hbm, xyzw_hbm,
             idx_v, emb_v, xyz_v, pers_v, cam_v, sem):
    wid = lax.axis_index("s") * 2 + lax.axis_index("c")
    wbase = wid * ROWS_PER_W

    pltpu.sync_copy(cam_hbm, cam_v)
    r = [_bcast(cam_v, k) for k in range(9)]
    cpx = _bcast(cam_v, 9)
    cpy = _bcast(cam_v, 10)
    cpz = _bcast(cam_v, 11)

    def group(g, carry):
        base = wbase + g * G
        pltpu.sync_copy(pidx_hbm.at[pl.ds(base, G)], idx_v)
        for j in range(NCH):
            sl = pl.ds(j * CH, CH)
            pltpu.async_copy(emb_hbm.at[idx_v.at[sl]], emb_v.at[sl], sem).wait()
            pltpu.async_copy(xyz_hbm.at[idx_v.at[sl]], xyz_v.at[sl], sem).wait()
        iota = lax.iota(jnp.int32, L)
        c0i = jnp.full((L,), 0, jnp.int32)
        c1i = jnp.full((L,), 1, jnp.int32)
        c2i = jnp.full((L,), 2, jnp.int32)
        for v in range(G // L):
            rvec = iota + (v * L)
            x = plsc.load_gather(xyz_v, [rvec, c0i])
            y = plsc.load_gather(xyz_v, [rvec, c1i])
            z = plsc.load_gather(xyz_v, [rvec, c2i])
            sx = x - cpx
            sy = y - cpy
            sz = z - cpz
            c0 = r[0] * sx + r[3] * sy + r[6] * sz
            c1 = r[1] * sx + r[4] * sy + r[7] * sz
            c2 = r[2] * sx + r[5] * sy + r[8] * sz
            den = c2 + 1e-9
            plsc.store_scatter(pers_v, [rvec, c0i], c0 / den)
            plsc.store_scatter(pers_v, [rvec, c1i], c1 / den)
            plsc.store_scatter(pers_v, [rvec, c2i], c2)
        pltpu.sync_copy(emb_v, feats_hbm.at[pl.ds(base, G)])
        pltpu.sync_copy(xyz_v, xyzw_hbm.at[pl.ds(base, G)])
        pltpu.sync_copy(pers_v, pers_hbm.at[pl.ds(base, G)])
        return carry

    lax.fori_loop(0, NG, group, 0)


@jax.jit
def _sc_gather(points_embeding, xyz, pidx_flat, cam):
    f32 = jnp.float32
    run = pl.kernel(
        _sc_body,
        out_type=(
            jax.ShapeDtypeStruct((M, FEAT), f32),
            jax.ShapeDtypeStruct((M, 3), f32),
            jax.ShapeDtypeStruct((M, 3), f32),
        ),
        mesh=plsc.VectorSubcoreMesh(core_axis_name="c", subcore_axis_name="s"),
        scratch_types=[
            pltpu.VMEM((G,), jnp.int32),
            pltpu.VMEM((G, FEAT), f32),
            pltpu.VMEM((G, 3), f32),
            pltpu.VMEM((G, 3), f32),
            pltpu.VMEM((L,), f32),
            pltpu.SemaphoreType.DMA,
        ],
    )
    return run(points_embeding, xyz, pidx_flat, cam)


def kernel(xyz, points_embeding, camrotc2w, campos, sample_pidx):
    pidx_flat = sample_pidx.reshape(-1).astype(jnp.int32)
    cam = jnp.concatenate(
        [camrotc2w.reshape(9), campos.reshape(3),
         jnp.zeros((4,), jnp.float32)]).astype(jnp.float32)
    feats, pers, xyzw = _sc_gather(points_embeding, xyz, pidx_flat, cam)
    sample_pnt_mask = sample_pidx >= 0
    Rw2c = jnp.eye(3, dtype=xyz.dtype)
    return (feats.reshape(B, R, SR, K, FEAT),
            pers.reshape(B, R, SR, K, 3),
            xyzw.reshape(B, R, SR, K, 3),
            sample_pnt_mask,
            Rw2c)


# speedup vs baseline: 1.0111x; 1.0111x over previous
import functools

import jax
import jax.numpy as jnp
from jax import lax
from jax.experimental import pallas as pl
from jax.experimental.pallas import tpu as pltpu
from jax.experimental.pallas import tpu_sc as plsc

N = 500000
FEAT = 32
B, R, SR, K = 1, 4096, 24, 8
M = B * R * SR * K
NW = 32
ROWS_PER_W = M // NW
G = 512
NG = ROWS_PER_W // G
CH = 128
NCH = G // CH
L = 16
XP = 8


def _bcast(cam_v, k):
    return plsc.load_gather(cam_v, [jnp.full((L,), k, jnp.int32)])


def _sc_body(emb_hbm, xyz_hbm, pidx_hbm, cam_hbm,
             feats_hbm, pers_hbm, xyzw_hbm,
             idx_v, emb_v, xyz_v, xyzw_v, pers_v, cam_v, sem):
    wid = lax.axis_index("s") * 2 + lax.axis_index("c")
    wbase = wid * ROWS_PER_W

    pltpu.sync_copy(cam_hbm, cam_v)
    r = [_bcast(cam_v, k + 1) for k in range(9)]
    cpx = _bcast(cam_v, 10)
    cpy = _bcast(cam_v, 11)
    cpz = _bcast(cam_v, 12)

    def group(g, carry):
        base = wbase + g * G
        pltpu.sync_copy(pidx_hbm.at[pl.ds(base, G)], idx_v)
        for j in range(NCH):
            sl = pl.ds(j * CH, CH)
            pltpu.async_copy(emb_hbm.at[idx_v.at[sl]], emb_v.at[sl], sem).wait()
            pltpu.async_copy(xyz_hbm.at[idx_v.at[sl]], xyz_v.at[sl], sem).wait()
        iota = lax.iota(jnp.int32, L)
        c0i = jnp.full((L,), 0, jnp.int32)
        c1i = jnp.full((L,), 1, jnp.int32)
        c2i = jnp.full((L,), 2, jnp.int32)
        for v in range(G // L):
            rvec = iota + (v * L)
            x = plsc.load_gather(xyz_v, [rvec, c0i])
            y = plsc.load_gather(xyz_v, [rvec, c1i])
            z = plsc.load_gather(xyz_v, [rvec, c2i])
            plsc.store_scatter(xyzw_v, [rvec, c0i], x)
            plsc.store_scatter(xyzw_v, [rvec, c1i], y)
            plsc.store_scatter(xyzw_v, [rvec, c2i], z)
            sx = x - cpx
            sy = y - cpy
            sz = z - cpz
            c0 = r[0] * sx + r[3] * sy + r[6] * sz
            c1 = r[1] * sx + r[4] * sy + r[7] * sz
            c2 = r[2] * sx + r[5] * sy + r[8] * sz
            den = c2 + 1e-9
            plsc.store_scatter(pers_v, [rvec, c0i], c0 / den)
            plsc.store_scatter(pers_v, [rvec, c1i], c1 / den)
            plsc.store_scatter(pers_v, [rvec, c2i], c2)
        pltpu.sync_copy(emb_v, feats_hbm.at[pl.ds(base, G)])
        pltpu.sync_copy(xyzw_v, xyzw_hbm.at[pl.ds(base, G)])
        pltpu.sync_copy(pers_v, pers_hbm.at[pl.ds(base, G)])
        return carry

    lax.fori_loop(0, NG, group, 0)


@jax.jit
def _sc_gather(points_embeding, xyz, pidx_flat, cam):
    f32 = jnp.float32
    run = pl.kernel(
        _sc_body,
        out_type=(
            jax.ShapeDtypeStruct((M, FEAT), f32),
            jax.ShapeDtypeStruct((M, 3), f32),
            jax.ShapeDtypeStruct((M, 3), f32),
        ),
        mesh=plsc.VectorSubcoreMesh(core_axis_name="c", subcore_axis_name="s"),
        compiler_params=pltpu.CompilerParams(
            needs_layout_passes=False, use_tc_tiling_on_sc=False),
        scratch_types=[
            pltpu.VMEM((G,), jnp.int32),
            pltpu.VMEM((G, FEAT), f32),
            pltpu.VMEM((G, XP), f32),
            pltpu.VMEM((G, 3), f32),
            pltpu.VMEM((G, 3), f32),
            pltpu.VMEM((L,), f32),
            pltpu.SemaphoreType.DMA,
        ],
    )
    return run(points_embeding, xyz, pidx_flat, cam)


def kernel(xyz, points_embeding, camrotc2w, campos, sample_pidx):
    pidx_flat = sample_pidx.reshape(-1).astype(jnp.int32)
    cam = jnp.concatenate(
        [jnp.zeros((1,), jnp.float32), camrotc2w.reshape(9),
         campos.reshape(3), jnp.zeros((3,), jnp.float32)]).astype(jnp.float32)
    xyz_pad = jnp.pad(xyz, ((0, 0), (0, XP - 3)))
    feats, pers, xyzw = _sc_gather(points_embeding, xyz_pad, pidx_flat, cam)
    sample_pnt_mask = sample_pidx >= 0
    Rw2c = jnp.eye(3, dtype=xyz.dtype)
    return (feats.reshape(B, R, SR, K, FEAT),
            pers.reshape(B, R, SR, K, 3),
            xyzw.reshape(B, R, SR, K, 3),
            sample_pnt_mask,
            Rw2c)


# speedup vs baseline: 1.2314x; 1.2178x over previous
import functools

import jax
import jax.numpy as jnp
from jax import lax
from jax.experimental import pallas as pl
from jax.experimental.pallas import tpu as pltpu
from jax.experimental.pallas import tpu_sc as plsc

N = 500000
FEAT = 32
B, R, SR, K = 1, 4096, 24, 8
M = B * R * SR * K
NW = 32
U = 1024
RT = R // 128
NU = SR * RT
UPW = NU // NW
CH = 128
NCH = U // CH
L = 16
XP = 8
FT = FEAT // 8


def _bcast(cam_v, k):
    return plsc.load_gather(cam_v, [jnp.full((L,), k, jnp.int32)])


def _sc_body(emb_hbm, xyz_hbm, pidx_hbm, cam_hbm,
             feats_hbm, pers_hbm, xyzw_hbm,
             idx_v, emb_v, xyz_v, feats_t, pers_t, xyzw_t, cam_v,
             sem, sem2, sem3):
    wid = lax.axis_index("s") * 2 + lax.axis_index("c")

    pltpu.sync_copy(cam_hbm, cam_v)
    r = [_bcast(cam_v, k + 1) for k in range(9)]
    cpx = _bcast(cam_v, 10)
    cpy = _bcast(cam_v, 11)
    cpz = _bcast(cam_v, 12)
    iota = lax.iota(jnp.int32, L)
    c0i = jnp.full((L,), 0, jnp.int32)
    c1i = jnp.full((L,), 1, jnp.int32)
    c2i = jnp.full((L,), 2, jnp.int32)

    def unit(i, carry):
        u = wid * UPW + i
        sr = u // RT
        c = u % RT
        pltpu.sync_copy(pidx_hbm.at[pl.ds(u * U, U)], idx_v)
        cps = []
        for j in range(NCH):
            sl = pl.ds(j * CH, CH)
            cps.append(
                pltpu.async_copy(emb_hbm.at[idx_v.at[sl]], emb_v.at[sl], sem))
            cps.append(
                pltpu.async_copy(xyz_hbm.at[idx_v.at[sl]], xyz_v.at[sl], sem2))
        for cp in cps:
            cp.wait()

        def xform(v, carry):
            rvec = iota + v * L
            sl16 = pl.ds(v * L, L)
            x = plsc.load_gather(xyz_v, [rvec, c0i])
            y = plsc.load_gather(xyz_v, [rvec, c1i])
            z = plsc.load_gather(xyz_v, [rvec, c2i])
            xyzw_t[0, sl16] = x
            xyzw_t[1, sl16] = y
            xyzw_t[2, sl16] = z
            sx = x - cpx
            sy = y - cpy
            sz = z - cpz
            v0 = r[0] * sx + r[3] * sy + r[6] * sz
            v1 = r[1] * sx + r[4] * sy + r[7] * sz
            v2 = r[2] * sx + r[5] * sy + r[8] * sz
            den = v2 + 1e-9
            pers_t[0, sl16] = v0 / den
            pers_t[1, sl16] = v1 / den
            pers_t[2, sl16] = v2
            return carry

        lax.fori_loop(0, U // L, xform, 0)

        def tpose(q, carry):
            k = q >> 5
            t = (q >> 3) & 3
            fm = q & 7
            col = jnp.full((L,), t * 8 + fm, jnp.int32)
            row0 = k * 128
            dst = k * FT + t
            for j in range(8):
                g = plsc.load_gather(emb_v, [row0 + j * L + iota, col])
                feats_t[dst, pl.ds(fm * 128 + j * L, L)] = g
            return carry

        lax.fori_loop(0, 256, tpose, 0)

        outs = []
        for k in range(K):
            for t in range(FT):
                off = ((sr * K + k) * 128 + t * RT + c) * U
                outs.append(pltpu.async_copy(
                    feats_t.at[k * FT + t], feats_hbm.at[pl.ds(off, U)], sem3))
        for comp in range(3):
            off = ((sr * 3 + comp) * RT + c) * U
            outs.append(pltpu.async_copy(
                pers_t.at[comp], pers_hbm.at[pl.ds(off, U)], sem3))
            outs.append(pltpu.async_copy(
                xyzw_t.at[comp], xyzw_hbm.at[pl.ds(off, U)], sem3))
        for cp in outs:
            cp.wait()
        return carry

    lax.fori_loop(0, UPW, unit, 0)


@jax.jit
def _sc_gather(points_embeding, xyz_pad, pidx_tiles, cam):
    f32 = jnp.float32
    run = pl.kernel(
        _sc_body,
        out_type=(
            jax.ShapeDtypeStruct((M * FEAT,), f32),
            jax.ShapeDtypeStruct((M * 3,), f32),
            jax.ShapeDtypeStruct((M * 3,), f32),
        ),
        mesh=plsc.VectorSubcoreMesh(core_axis_name="c", subcore_axis_name="s"),
        compiler_params=pltpu.CompilerParams(
            needs_layout_passes=False, use_tc_tiling_on_sc=False),
        scratch_types=[
            pltpu.VMEM((U,), jnp.int32),
            pltpu.VMEM((U, FEAT), f32),
            pltpu.VMEM((U, XP), f32),
            pltpu.VMEM((K * FT, 128 * 8), f32),
            pltpu.VMEM((3, U), f32),
            pltpu.VMEM((3, U), f32),
            pltpu.VMEM((L,), f32),
            pltpu.SemaphoreType.DMA,
            pltpu.SemaphoreType.DMA,
            pltpu.SemaphoreType.DMA,
        ],
    )
    return run(points_embeding, xyz_pad, pidx_tiles, cam)


def kernel(xyz, points_embeding, camrotc2w, campos, sample_pidx):
    pidx_tiles = (sample_pidx.reshape(RT, 128, SR, K)
                  .transpose(2, 0, 3, 1).reshape(-1).astype(jnp.int32))
    cam = jnp.concatenate(
        [jnp.zeros((1,), jnp.float32), camrotc2w.reshape(9),
         campos.reshape(3), jnp.zeros((3,), jnp.float32)]).astype(jnp.float32)
    xyz_pad = jnp.pad(xyz, ((0, 0), (0, XP - 3)))
    feats_img, pers_img, xyzw_img = _sc_gather(
        points_embeding, xyz_pad, pidx_tiles, cam)
    feats = (feats_img.reshape(SR, K, FT, RT, 8, 128)
             .transpose(3, 5, 0, 1, 2, 4).reshape(1, R, SR, K, FEAT))
    pers = (pers_img.reshape(SR, 3, RT, K, 128)
            .transpose(2, 4, 0, 3, 1).reshape(1, R, SR, K, 3))
    xyzw = (xyzw_img.reshape(SR, 3, RT, K, 128)
            .transpose(2, 4, 0, 3, 1).reshape(1, R, SR, K, 3))
    sample_pnt_mask = sample_pidx >= 0
    Rw2c = jnp.eye(3, dtype=xyz.dtype)
    return (feats, pers, xyzw, sample_pnt_mask, Rw2c)


# speedup vs baseline: 1.2317x; 1.0002x over previous
import functools

import jax
import jax.numpy as jnp
from jax import lax
from jax.experimental import pallas as pl
from jax.experimental.pallas import tpu as pltpu
from jax.experimental.pallas import tpu_sc as plsc

N = 500000
FEAT = 32
B, R, SR, K = 1, 4096, 24, 8
M = B * R * SR * K
NW = 32
U = 1024
RT = R // 128
NU = SR * RT
UPW = NU // NW
CH = 128
NCH = U // CH
L = 16
XP = 8
FT = FEAT // 8


def _bcast(cam_v, k):
    return plsc.load_gather(cam_v, [jnp.full((L,), k, jnp.int32)])


def _sc_body(emb_hbm, xyz_hbm, pidx_hbm, cam_hbm,
             feats_hbm, pers_hbm, xyzw_hbm,
             idx_v, emb_v, xyz_v, feats_t, pers_t, xyzw_t, cam_v,
             sem, sem2, sem3):
    wid = lax.axis_index("s") * 2 + lax.axis_index("c")

    pltpu.sync_copy(cam_hbm, cam_v)
    r = [_bcast(cam_v, k + 1) for k in range(9)]
    cpx = _bcast(cam_v, 10)
    cpy = _bcast(cam_v, 11)
    cpz = _bcast(cam_v, 12)
    iota = lax.iota(jnp.int32, L)
    c0i = jnp.full((L,), 0, jnp.int32)
    c1i = jnp.full((L,), 1, jnp.int32)
    c2i = jnp.full((L,), 2, jnp.int32)

    def unit(i, carry):
        u = wid * UPW + i
        sr = u // RT
        c = u % RT
        pltpu.sync_copy(pidx_hbm.at[pl.ds(u * U, U)], idx_v)
        cps = []
        for j in range(NCH):
            sl = pl.ds(j * CH, CH)
            cps.append(
                pltpu.async_copy(emb_hbm.at[idx_v.at[sl]], emb_v.at[sl], sem))
            cps.append(
                pltpu.async_copy(xyz_hbm.at[idx_v.at[sl]], xyz_v.at[sl], sem2))
        for cp in cps:
            cp.wait()

        def xform(v, carry):
            rvec = iota + v * L
            sl16 = pl.ds(v * L, L)
            x = plsc.load_gather(xyz_v, [rvec, c0i])
            y = plsc.load_gather(xyz_v, [rvec, c1i])
            z = plsc.load_gather(xyz_v, [rvec, c2i])
            xyzw_t[0, sl16] = x
            xyzw_t[1, sl16] = y
            xyzw_t[2, sl16] = z
            sx = x - cpx
            sy = y - cpy
            sz = z - cpz
            v0 = r[0] * sx + r[3] * sy + r[6] * sz
            v1 = r[1] * sx + r[4] * sy + r[7] * sz
            v2 = r[2] * sx + r[5] * sy + r[8] * sz
            den = v2 + 1e-9
            pers_t[0, sl16] = v0 / den
            pers_t[1, sl16] = v1 / den
            pers_t[2, sl16] = v2
            return carry

        lax.fori_loop(0, U // L, xform, 0)

        def tpose(q, carry):
            k = q >> 5
            t = (q >> 3) & 3
            fm = q & 7
            col = jnp.full((L,), t * 8 + fm, jnp.int32)
            row0 = k * 128
            dst = k * FT + t
            for j in range(8):
                g = plsc.load_gather(emb_v, [row0 + j * L + iota, col])
                feats_t[dst, pl.ds(fm * 128 + j * L, L)] = g
            return carry

        lax.fori_loop(0, 256, tpose, 0)

        outs = []
        for k in range(K):
            for t in range(FT):
                off = ((sr * K + k) * 128 + t * RT + c) * U
                outs.append(pltpu.async_copy(
                    feats_t.at[k * FT + t], feats_hbm.at[pl.ds(off, U)], sem3))
        for comp in range(3):
            off = ((sr * 3 + comp) * RT + c) * U
            outs.append(pltpu.async_copy(
                pers_t.at[comp], pers_hbm.at[pl.ds(off, U)], sem3))
            outs.append(pltpu.async_copy(
                xyzw_t.at[comp], xyzw_hbm.at[pl.ds(off, U)], sem3))
        for cp in outs:
            cp.wait()
        return carry

    lax.fori_loop(0, UPW, unit, 0)


@jax.jit
def _sc_gather(points_embeding, xyz_pad, pidx_tiles, cam):
    f32 = jnp.float32
    run = pl.kernel(
        _sc_body,
        out_type=(
            jax.ShapeDtypeStruct((M * FEAT,), f32),
            jax.ShapeDtypeStruct((M * 3,), f32),
            jax.ShapeDtypeStruct((M * 3,), f32),
        ),
        mesh=plsc.VectorSubcoreMesh(core_axis_name="c", subcore_axis_name="s"),
        compiler_params=pltpu.CompilerParams(
            needs_layout_passes=False, use_tc_tiling_on_sc=False),
        scratch_types=[
            pltpu.VMEM((U,), jnp.int32),
            pltpu.VMEM((U, FEAT), f32),
            pltpu.VMEM((U, XP), f32),
            pltpu.VMEM((K * FT, 128 * 8), f32),
            pltpu.VMEM((3, U), f32),
            pltpu.VMEM((3, U), f32),
            pltpu.VMEM((L,), f32),
            pltpu.SemaphoreType.DMA,
            pltpu.SemaphoreType.DMA,
            pltpu.SemaphoreType.DMA,
        ],
    )
    return run(points_embeding, xyz_pad, pidx_tiles, cam)


def kernel(xyz, points_embeding, camrotc2w, campos, sample_pidx):
    pidx_tiles = (sample_pidx.reshape(RT, 128, SR, K)
                  .transpose(2, 0, 3, 1).reshape(-1).astype(jnp.int32))
    cam = jnp.concatenate(
        [jnp.zeros((1,), jnp.float32), camrotc2w.reshape(9),
         campos.reshape(3), jnp.zeros((3,), jnp.float32)]).astype(jnp.float32)
    xyz_pad = jnp.pad(xyz, ((0, 0), (0, XP - 3)))
    feats_img, pers_img, xyzw_img = _sc_gather(
        points_embeding, xyz_pad, pidx_tiles, cam)
    feats = (feats_img.reshape(SR, K, FT, RT, 8, 128)
             .transpose(3, 5, 0, 1, 2, 4).reshape(1, R, SR, K, FEAT)
             + jnp.float32(0.0))
    pers = (pers_img.reshape(SR, 3, RT, K, 128)
            .transpose(2, 4, 0, 3, 1).reshape(1, R, SR, K, 3))
    xyzw = (xyzw_img.reshape(SR, 3, RT, K, 128)
            .transpose(2, 4, 0, 3, 1).reshape(1, R, SR, K, 3))
    sample_pnt_mask = sample_pidx >= 0
    Rw2c = jnp.eye(3, dtype=xyz.dtype)
    return (feats, pers, xyzw, sample_pnt_mask, Rw2c)
